# Initial kernel scaffold; baseline (speedup 1.0000x reference)
import jax
import jax.numpy as jnp
from jax.experimental import pallas as pl


def kernel(coord, emb_token, emb_value, W_proj):
    raise NotImplementedError("write your pallas kernel here")


# speedup vs baseline: 9.8299x; 9.8299x over previous
import functools

import jax
import jax.numpy as jnp
from jax import lax
from jax.experimental import pallas as pl
from jax.experimental.pallas import tpu as pltpu
from jax.experimental.pallas import tpu_sc as plsc

A = 3
E = 32
M = 128
V = 1024 * A
N = 4096
S = 200
ROWS = S + A
HALF = S // 2

NC, NS = 2, 16
NW = NC * NS
N_PER_W = N // NW


def _table_body(ev_ref, w_ref, t_ref):
    ev = ev_ref[...]
    for a in range(A):
        t_ref[a] = jnp.dot(ev, w_ref[a], preferred_element_type=jnp.float32)


def _make_table(emb_value, w_r):
    return pl.pallas_call(
        _table_body,
        out_shape=jax.ShapeDtypeStruct((A, V, M), jnp.float32),
    )(emb_value, w_r)


def _sc_gather(table, emb_token, idx):
    mesh = plsc.VectorSubcoreMesh(core_axis_name="c", subcore_axis_name="s")

    @functools.partial(
        pl.kernel,
        out_type=jax.ShapeDtypeStruct((N, ROWS, M), jnp.float32),
        mesh=mesh,
        scratch_types=[
            pltpu.VMEM((2 * A, HALF), jnp.int32),
            pltpu.VMEM((ROWS, M), jnp.float32),
            pltpu.SemaphoreType.DMA,
            pltpu.SemaphoreType.DMA,
        ],
    )
    def k(table_hbm, tok_hbm, idx_hbm, out_hbm, idxb, buf, gsem, osem):
        wid = lax.axis_index("s") * NC + lax.axis_index("c")
        base_n = wid * N_PER_W
        pltpu.sync_copy(tok_hbm, buf.at[pl.ds(0, A)])

        def loop_n(i, carry):
            n = base_n + i
            pltpu.sync_copy(idx_hbm.at[n], idxb)
            for h in range(2):
                dst = buf.at[pl.ds(A + HALF * h, HALF)]
                pltpu.async_copy(table_hbm.at[idxb.at[A * h + 0]], dst, gsem).wait()
                pltpu.async_copy(table_hbm.at[idxb.at[A * h + 1]], dst, gsem, add=True).wait()
                pltpu.async_copy(table_hbm.at[idxb.at[A * h + 2]], dst, gsem, add=True).wait()
            pltpu.async_copy(buf, out_hbm.at[n], osem).wait()
            return carry

        lax.fori_loop(0, N_PER_W, loop_n, 0)

    return k(table, emb_token, idx)


def kernel(coord, emb_token, emb_value, W_proj):
    w_r = W_proj.reshape(M, A, E).transpose(1, 2, 0)
    table = _make_table(emb_value, w_r).reshape(A * V, M)

    coord32 = coord.astype(jnp.int32)
    idx = coord32 + (jnp.arange(A, dtype=jnp.int32) * V)[None, None, :]
    idx = idx.reshape(N, 2, HALF, A).transpose(0, 1, 3, 2).reshape(N, 2 * A, HALF)

    return _sc_gather(table, emb_token, idx)
